# baseline (device time: 46238 ns/iter reference)
import jax
import jax.numpy as jnp
from jax import lax
from jax.experimental import pallas as pl
from jax.experimental.pallas import tpu as pltpu

N_DEV = 4
K = 4
WIRE_DTYPE = jnp.bfloat16


def kernel(x):
    m, n = x.shape
    chunk = m // N_DEV
    n2 = n // 2
    sub = n2 // K
    n_hops = 2 * (N_DEV - 1)
    n_sems = n_hops * 2 * K

    def body(x_ref, out_ref, rs_r, rs_l, st_r, st_l, ag_r, ag_l,
             send_sems, recv_sems):
        my = lax.axis_index("i")
        left = (my + N_DEV - 1) % N_DEV
        right = (my + 1) % N_DEV

        barrier_sem = pltpu.get_barrier_semaphore()
        for nbr in [left, right]:
            pl.semaphore_signal(
                barrier_sem, inc=1,
                device_id=(nbr,), device_id_type=pl.DeviceIdType.MESH,
            )
        pl.semaphore_wait(barrier_sem, 2)

        def dev(d):
            return right if d == 0 else left

        def gcol(d, c):
            return d * n2 + c * sub

        def rs_buf(d):
            return rs_r if d == 0 else rs_l

        def st_buf(d):
            return st_r if d == 0 else st_l

        def ag_buf(d):
            return ag_r if d == 0 else ag_l

        def rs_recv_chunk(d, h):
            return ((my + N_DEV - h - 1) if d == 0 else (my + h + 1)) % N_DEV

        def owned_chunk(d):
            return ((my + 1) if d == 0 else (my + N_DEV - 1)) % N_DEV

        def ag_send_chunk(d, t):
            return ((my + 1 + N_DEV - t) if d == 0 else (my + N_DEV - 1 + t)) % N_DEV

        def ag_recv_chunk(d, t):
            return ((my + N_DEV - t) if d == 0 else (my + t)) % N_DEV

        def sem(h, d, c):
            return (h * 2 + d) * K + c

        def copy(src, dst, h, d, c):
            return pltpu.make_async_remote_copy(
                src_ref=src, dst_ref=dst,
                send_sem=send_sems.at[sem(h, d, c)],
                recv_sem=recv_sems.at[sem(h, d, c)],
                device_id=(dev(d),), device_id_type=pl.DeviceIdType.MESH,
            )

        started = []
        pend = {}

        def start(rdma, d, c):
            rdma.start()
            started.append(rdma)
            pend[(d, c)] = rdma

        for c in range(K):
            for d in range(2):
                cc = pl.ds(c * sub, sub)
                st_buf(d)[:, cc] = x_ref[
                    pl.ds(my * chunk, chunk), pl.ds(gcol(d, c), sub)
                ].astype(WIRE_DTYPE)
                r = copy(
                    st_buf(d).at[:, cc],
                    rs_buf(d).at[0, :, cc],
                    0, d, c,
                )
                start(r, d, c)

        for h in range(1, N_DEV - 1):
            for c in range(K):
                for d in range(2):
                    cc = pl.ds(c * sub, sub)
                    pend[(d, c)].wait_recv()
                    ch = rs_recv_chunk(d, h - 1)
                    rs_buf(d)[h - 1, :, cc] = (
                        rs_buf(d)[h - 1, :, cc].astype(jnp.float32)
                        + x_ref[pl.ds(ch * chunk, chunk), pl.ds(gcol(d, c), sub)]
                    ).astype(WIRE_DTYPE)
                    r = copy(
                        rs_buf(d).at[h - 1, :, cc],
                        rs_buf(d).at[h, :, cc],
                        h, d, c,
                    )
                    start(r, d, c)

        for c in range(K):
            for d in range(2):
                cc = pl.ds(c * sub, sub)
                pend[(d, c)].wait_recv()
                q = owned_chunk(d)
                red = (
                    rs_buf(d)[N_DEV - 2, :, cc].astype(jnp.float32)
                    + x_ref[pl.ds(q * chunk, chunk), pl.ds(gcol(d, c), sub)]
                )
                out_ref[pl.ds(q * chunk, chunk), pl.ds(gcol(d, c), sub)] = red
                ag_buf(d)[q, :, cc] = red.astype(WIRE_DTYPE)
                r = copy(
                    ag_buf(d).at[q, :, cc],
                    ag_buf(d).at[q, :, cc],
                    N_DEV - 1, d, c,
                )
                start(r, d, c)

        for t in range(1, N_DEV - 1):
            for c in range(K):
                for d in range(2):
                    cc = pl.ds(c * sub, sub)
                    pend[(d, c)].wait_recv()
                    rc = ag_recv_chunk(d, t - 1)
                    ch = ag_send_chunk(d, t)
                    r = copy(
                        ag_buf(d).at[ch, :, cc],
                        ag_buf(d).at[ch, :, cc],
                        N_DEV - 1 + t, d, c,
                    )
                    start(r, d, c)
                    out_ref[pl.ds(rc * chunk, chunk), pl.ds(gcol(d, c), sub)] = (
                        ag_buf(d)[rc, :, cc].astype(jnp.float32)
                    )

        for c in range(K):
            for d in range(2):
                cc = pl.ds(c * sub, sub)
                pend[(d, c)].wait_recv()
                rc = ag_recv_chunk(d, N_DEV - 2)
                out_ref[pl.ds(rc * chunk, chunk), pl.ds(gcol(d, c), sub)] = (
                    ag_buf(d)[rc, :, cc].astype(jnp.float32)
                )
        for r in started:
            r.wait_send()

    return pl.pallas_call(
        body,
        out_shape=jax.ShapeDtypeStruct((m, n), x.dtype),
        in_specs=[pl.BlockSpec(memory_space=pltpu.VMEM)],
        out_specs=pl.BlockSpec(memory_space=pltpu.VMEM),
        scratch_shapes=[
            pltpu.VMEM((N_DEV - 1, chunk, n2), WIRE_DTYPE),
            pltpu.VMEM((N_DEV - 1, chunk, n2), WIRE_DTYPE),
            pltpu.VMEM((chunk, n2), WIRE_DTYPE),
            pltpu.VMEM((chunk, n2), WIRE_DTYPE),
            pltpu.VMEM((N_DEV, chunk, n2), WIRE_DTYPE),
            pltpu.VMEM((N_DEV, chunk, n2), WIRE_DTYPE),
            pltpu.SemaphoreType.DMA((n_sems,)),
            pltpu.SemaphoreType.DMA((n_sems,)),
        ],
        compiler_params=pltpu.CompilerParams(collective_id=0),
    )(x)


# device time: 45845 ns/iter; 1.0086x vs baseline; 1.0086x over previous
import jax
import jax.numpy as jnp
from jax import lax
from jax.experimental import pallas as pl
from jax.experimental.pallas import tpu as pltpu

N_DEV = 4
K = 2
WIRE_DTYPE = jnp.bfloat16


def kernel(x):
    m, n = x.shape
    chunk = m // N_DEV
    n2 = n // 2
    sub = n2 // K
    n_hops = 2 * (N_DEV - 1)
    n_sems = n_hops * 2 * K

    def body(x_ref, out_ref, rs_r, rs_l, st_r, st_l, ag_r, ag_l,
             send_sems, recv_sems):
        my = lax.axis_index("i")
        left = (my + N_DEV - 1) % N_DEV
        right = (my + 1) % N_DEV

        def dev(d):
            return right if d == 0 else left

        def gcol(d, c):
            return d * n2 + c * sub

        def rs_buf(d):
            return rs_r if d == 0 else rs_l

        def st_buf(d):
            return st_r if d == 0 else st_l

        def ag_buf(d):
            return ag_r if d == 0 else ag_l

        def rs_recv_chunk(d, h):
            return ((my + N_DEV - h - 1) if d == 0 else (my + h + 1)) % N_DEV

        def owned_chunk(d):
            return ((my + 1) if d == 0 else (my + N_DEV - 1)) % N_DEV

        def ag_send_chunk(d, t):
            return ((my + 1 + N_DEV - t) if d == 0 else (my + N_DEV - 1 + t)) % N_DEV

        def ag_recv_chunk(d, t):
            return ((my + N_DEV - t) if d == 0 else (my + t)) % N_DEV

        def sem(h, d, c):
            return (h * 2 + d) * K + c

        def copy(src, dst, h, d, c):
            return pltpu.make_async_remote_copy(
                src_ref=src, dst_ref=dst,
                send_sem=send_sems.at[sem(h, d, c)],
                recv_sem=recv_sems.at[sem(h, d, c)],
                device_id=(dev(d),), device_id_type=pl.DeviceIdType.MESH,
            )

        started = []
        pend = {}

        def start(rdma, d, c):
            rdma.start()
            started.append(rdma)
            pend[(d, c)] = rdma

        for d in range(2):
            st_buf(d)[:, :] = x_ref[
                pl.ds(my * chunk, chunk), pl.ds(d * n2, n2)
            ].astype(WIRE_DTYPE)

        barrier_sem = pltpu.get_barrier_semaphore()
        for nbr in [left, right]:
            pl.semaphore_signal(
                barrier_sem, inc=1,
                device_id=(nbr,), device_id_type=pl.DeviceIdType.MESH,
            )
        pl.semaphore_wait(barrier_sem, 2)

        for c in range(K):
            for d in range(2):
                cc = pl.ds(c * sub, sub)
                r = copy(
                    st_buf(d).at[:, cc],
                    rs_buf(d).at[0, :, cc],
                    0, d, c,
                )
                start(r, d, c)

        for h in range(1, N_DEV - 1):
            for c in range(K):
                for d in range(2):
                    cc = pl.ds(c * sub, sub)
                    pend[(d, c)].wait_recv()
                    ch = rs_recv_chunk(d, h - 1)
                    rs_buf(d)[h - 1, :, cc] = (
                        rs_buf(d)[h - 1, :, cc].astype(jnp.float32)
                        + x_ref[pl.ds(ch * chunk, chunk), pl.ds(gcol(d, c), sub)]
                    ).astype(WIRE_DTYPE)
                    r = copy(
                        rs_buf(d).at[h - 1, :, cc],
                        rs_buf(d).at[h, :, cc],
                        h, d, c,
                    )
                    start(r, d, c)

        for c in range(K):
            for d in range(2):
                cc = pl.ds(c * sub, sub)
                pend[(d, c)].wait_recv()
                q = owned_chunk(d)
                red = (
                    rs_buf(d)[N_DEV - 2, :, cc].astype(jnp.float32)
                    + x_ref[pl.ds(q * chunk, chunk), pl.ds(gcol(d, c), sub)]
                )
                out_ref[pl.ds(q * chunk, chunk), pl.ds(gcol(d, c), sub)] = red
                ag_buf(d)[q, :, cc] = red.astype(WIRE_DTYPE)
                r = copy(
                    ag_buf(d).at[q, :, cc],
                    ag_buf(d).at[q, :, cc],
                    N_DEV - 1, d, c,
                )
                start(r, d, c)

        for t in range(1, N_DEV - 1):
            for c in range(K):
                for d in range(2):
                    cc = pl.ds(c * sub, sub)
                    pend[(d, c)].wait_recv()
                    rc = ag_recv_chunk(d, t - 1)
                    ch = ag_send_chunk(d, t)
                    r = copy(
                        ag_buf(d).at[ch, :, cc],
                        ag_buf(d).at[ch, :, cc],
                        N_DEV - 1 + t, d, c,
                    )
                    start(r, d, c)
                    out_ref[pl.ds(rc * chunk, chunk), pl.ds(gcol(d, c), sub)] = (
                        ag_buf(d)[rc, :, cc].astype(jnp.float32)
                    )

        for c in range(K):
            for d in range(2):
                cc = pl.ds(c * sub, sub)
                pend[(d, c)].wait_recv()
                rc = ag_recv_chunk(d, N_DEV - 2)
                out_ref[pl.ds(rc * chunk, chunk), pl.ds(gcol(d, c), sub)] = (
                    ag_buf(d)[rc, :, cc].astype(jnp.float32)
                )
        for r in started:
            r.wait_send()

    return pl.pallas_call(
        body,
        out_shape=jax.ShapeDtypeStruct((m, n), x.dtype),
        in_specs=[pl.BlockSpec(memory_space=pltpu.VMEM)],
        out_specs=pl.BlockSpec(memory_space=pltpu.VMEM),
        scratch_shapes=[
            pltpu.VMEM((N_DEV - 1, chunk, n2), WIRE_DTYPE),
            pltpu.VMEM((N_DEV - 1, chunk, n2), WIRE_DTYPE),
            pltpu.VMEM((chunk, n2), WIRE_DTYPE),
            pltpu.VMEM((chunk, n2), WIRE_DTYPE),
            pltpu.VMEM((N_DEV, chunk, n2), WIRE_DTYPE),
            pltpu.VMEM((N_DEV, chunk, n2), WIRE_DTYPE),
            pltpu.SemaphoreType.DMA((n_sems,)),
            pltpu.SemaphoreType.DMA((n_sems,)),
        ],
        compiler_params=pltpu.CompilerParams(collective_id=0),
    )(x)


# device time: 44824 ns/iter; 1.0315x vs baseline; 1.0228x over previous
import jax
import jax.numpy as jnp
from jax import lax
from jax.experimental import pallas as pl
from jax.experimental.pallas import tpu as pltpu

N_DEV = 4
K = 2
WIRE_DTYPE = jnp.bfloat16
F32 = jnp.float32

CHAIN_R, DIRECT_R, PARTIAL_R, AG_OWN_R_RIGHT, AG_OWN_R_LEFT, AG_FWD_R = range(6)
CHAIN_L, DIRECT_L, PARTIAL_L, AG_OWN_L_RIGHT, AG_OWN_L_LEFT, AG_FWD_L = range(6, 12)
N_TYPES = 12


def kernel(x):
    m, n = x.shape
    chunk = m // N_DEV
    n2 = n // 2
    sub = n2 // K
    n_sems = N_TYPES * K

    def body(x_ref, out_ref, st_a, st_b, st_c,
             in_chain_r, in_chain_l, in_direct_r, in_direct_l,
             in_partial_r, in_partial_l, st_partial_r, st_partial_l,
             ag, send_sems, recv_sems):
        my = lax.axis_index("i")
        left = (my + N_DEV - 1) % N_DEV
        right = (my + 1) % N_DEV
        p1 = (my + 1) % N_DEV
        m1 = (my + N_DEV - 1) % N_DEV
        p2 = (my + 2) % N_DEV

        def rcc(c):
            return pl.ds(c * sub, sub)

        def lcc(c):
            return pl.ds(n2 + c * sub, sub)

        st_a[:, :] = x_ref[pl.ds(p2 * chunk, chunk), :].astype(WIRE_DTYPE)
        st_b[:, :] = x_ref[pl.ds(p1 * chunk, chunk), pl.ds(0, n2)].astype(WIRE_DTYPE)
        st_c[:, :] = x_ref[pl.ds(m1 * chunk, chunk), pl.ds(n2, n2)].astype(WIRE_DTYPE)

        barrier_sem = pltpu.get_barrier_semaphore()
        for nbr in [left, right]:
            pl.semaphore_signal(
                barrier_sem, inc=1,
                device_id=(nbr,), device_id_type=pl.DeviceIdType.MESH,
            )
        pl.semaphore_wait(barrier_sem, 2)

        started = []
        pend = {}

        def send(src, dst, tid, c, dev_):
            r = pltpu.make_async_remote_copy(
                src_ref=src, dst_ref=dst,
                send_sem=send_sems.at[tid * K + c],
                recv_sem=recv_sems.at[tid * K + c],
                device_id=(dev_,), device_id_type=pl.DeviceIdType.MESH,
            )
            r.start()
            started.append(r)
            pend[(tid, c)] = r

        for c in range(K):
            send(st_a.at[:, rcc(c)], in_chain_r.at[:, rcc(c)], CHAIN_R, c, left)
            send(st_a.at[:, lcc(c)], in_chain_l.at[:, rcc(c)], CHAIN_L, c, right)
            send(st_b.at[:, rcc(c)], in_direct_r.at[:, rcc(c)], DIRECT_R, c, right)
            send(st_c.at[:, rcc(c)], in_direct_l.at[:, rcc(c)], DIRECT_L, c, left)

        for c in range(K):
            pend[(CHAIN_R, c)].wait_recv()
            st_partial_r[:, rcc(c)] = (
                in_chain_r[:, rcc(c)].astype(F32)
                + x_ref[pl.ds(m1 * chunk, chunk), rcc(c)]
            ).astype(WIRE_DTYPE)
            send(st_partial_r.at[:, rcc(c)], in_partial_r.at[:, rcc(c)],
                 PARTIAL_R, c, left)
            pend[(CHAIN_L, c)].wait_recv()
            st_partial_l[:, rcc(c)] = (
                in_chain_l[:, rcc(c)].astype(F32)
                + x_ref[pl.ds(p1 * chunk, chunk), lcc(c)]
            ).astype(WIRE_DTYPE)
            send(st_partial_l.at[:, rcc(c)], in_partial_l.at[:, rcc(c)],
                 PARTIAL_L, c, right)

        for c in range(K):
            pend[(PARTIAL_R, c)].wait_recv()
            pend[(DIRECT_R, c)].wait_recv()
            sum_r = (
                in_partial_r[:, rcc(c)].astype(F32)
                + in_direct_r[:, rcc(c)].astype(F32)
                + x_ref[pl.ds(my * chunk, chunk), rcc(c)]
            )
            out_ref[pl.ds(my * chunk, chunk), rcc(c)] = sum_r
            ag[my, :, rcc(c)] = sum_r.astype(WIRE_DTYPE)
            send(ag.at[my, :, rcc(c)], ag.at[my, :, rcc(c)],
                 AG_OWN_R_RIGHT, c, right)
            send(ag.at[my, :, rcc(c)], ag.at[my, :, rcc(c)],
                 AG_OWN_R_LEFT, c, left)

            pend[(PARTIAL_L, c)].wait_recv()
            pend[(DIRECT_L, c)].wait_recv()
            sum_l = (
                in_partial_l[:, rcc(c)].astype(F32)
                + in_direct_l[:, rcc(c)].astype(F32)
                + x_ref[pl.ds(my * chunk, chunk), lcc(c)]
            )
            out_ref[pl.ds(my * chunk, chunk), lcc(c)] = sum_l
            ag[my, :, lcc(c)] = sum_l.astype(WIRE_DTYPE)
            send(ag.at[my, :, lcc(c)], ag.at[my, :, lcc(c)],
                 AG_OWN_L_RIGHT, c, right)
            send(ag.at[my, :, lcc(c)], ag.at[my, :, lcc(c)],
                 AG_OWN_L_LEFT, c, left)

        for c in range(K):
            pend[(AG_OWN_R_RIGHT, c)].wait_recv()
            send(ag.at[m1, :, rcc(c)], ag.at[m1, :, rcc(c)], AG_FWD_R, c, right)
            pend[(AG_OWN_L_LEFT, c)].wait_recv()
            send(ag.at[p1, :, lcc(c)], ag.at[p1, :, lcc(c)], AG_FWD_L, c, left)
            out_ref[pl.ds(m1 * chunk, chunk), rcc(c)] = (
                ag[m1, :, rcc(c)].astype(F32)
            )
            out_ref[pl.ds(p1 * chunk, chunk), lcc(c)] = (
                ag[p1, :, lcc(c)].astype(F32)
            )

        for c in range(K):
            pend[(AG_OWN_R_LEFT, c)].wait_recv()
            out_ref[pl.ds(p1 * chunk, chunk), rcc(c)] = (
                ag[p1, :, rcc(c)].astype(F32)
            )
            pend[(AG_OWN_L_RIGHT, c)].wait_recv()
            out_ref[pl.ds(m1 * chunk, chunk), lcc(c)] = (
                ag[m1, :, lcc(c)].astype(F32)
            )
            pend[(AG_FWD_R, c)].wait_recv()
            out_ref[pl.ds(p2 * chunk, chunk), rcc(c)] = (
                ag[p2, :, rcc(c)].astype(F32)
            )
            pend[(AG_FWD_L, c)].wait_recv()
            out_ref[pl.ds(p2 * chunk, chunk), lcc(c)] = (
                ag[p2, :, lcc(c)].astype(F32)
            )

        for r in started:
            r.wait_send()

    return pl.pallas_call(
        body,
        out_shape=jax.ShapeDtypeStruct((m, n), x.dtype),
        in_specs=[pl.BlockSpec(memory_space=pltpu.VMEM)],
        out_specs=pl.BlockSpec(memory_space=pltpu.VMEM),
        scratch_shapes=[
            pltpu.VMEM((chunk, n), WIRE_DTYPE),
            pltpu.VMEM((chunk, n2), WIRE_DTYPE),
            pltpu.VMEM((chunk, n2), WIRE_DTYPE),
            pltpu.VMEM((chunk, n2), WIRE_DTYPE),
            pltpu.VMEM((chunk, n2), WIRE_DTYPE),
            pltpu.VMEM((chunk, n2), WIRE_DTYPE),
            pltpu.VMEM((chunk, n2), WIRE_DTYPE),
            pltpu.VMEM((chunk, n2), WIRE_DTYPE),
            pltpu.VMEM((chunk, n2), WIRE_DTYPE),
            pltpu.VMEM((chunk, n2), WIRE_DTYPE),
            pltpu.VMEM((chunk, n2), WIRE_DTYPE),
            pltpu.VMEM((N_DEV, chunk, n), WIRE_DTYPE),
            pltpu.SemaphoreType.DMA((n_sems,)),
            pltpu.SemaphoreType.DMA((n_sems,)),
        ],
        compiler_params=pltpu.CompilerParams(collective_id=0),
    )(x)
